# baseline (speedup 1.0000x reference)
import functools

import jax
import jax.numpy as jnp
from jax import lax
from jax.experimental import pallas as pl
from jax.experimental.pallas import tpu as pltpu
from jax.experimental.pallas import tpu_sc as plsc

VOCAB = 1000000
DIM = 64
BATCH = 16384
CTX = 20

NC = 2
NS = 16
NW = NC * NS
BPW = BATCH // NW
STEP = 32
NSTEP = BPW // STEP
NG = CTX * STEP // 128
NK = DIM // 16

W = 384
WP = W + 1
NCOL = 2604
TAILV = VOCAB - NCOL * W
MPT = (NCOL + NW - 1) // NW


def _relayout_body(wt_hbm, tail_hbm, out_hbm,
                   win_a, win_b, stage_a, stage_b, sem_a, sem_b, semw):
    wid = lax.axis_index("s") * NC + lax.axis_index("c")
    lane = lax.iota(jnp.int32, 16)
    rows_k = [lane + 16 * k for k in range(NK)]

    def fire_reads(j, win, sem):
        v0 = pl.multiple_of(j * W, W)
        for g in range(8):
            pltpu.make_async_copy(
                wt_hbm.at[pl.ds(g * 8, 8), pl.ds(v0, W)],
                win.at[pl.ds(g * 8, 8), pl.ds(0, W)], sem).start()

    def transpose(j, win, stage, sem):
        for g in range(8):
            pltpu.make_async_copy(
                wt_hbm.at[pl.ds(0, 8), pl.ds(0, W)],
                win.at[pl.ds(0, 8), pl.ds(0, W)], sem).wait()

        def v_body(v, carry):
            p = (v & 1) * 64
            r = v >> 1
            col = jnp.full((16,), 0, jnp.int32) + v
            for k in range(NK):
                chunk = plsc.load_gather(win, [rows_k[k], col])
                stage[r, pl.ds(p + k * 16, 16)] = chunk
            return carry

        lax.fori_loop(0, W, v_body, 0)
        pltpu.make_async_copy(
            stage, out_hbm.at[pl.ds(pl.multiple_of(j * (W // 2), W // 2),
                                    W // 2)], semw).start()

    def drain_write(stage):
        pltpu.make_async_copy(
            stage, out_hbm.at[pl.ds(0, W // 2)], semw).wait()

    def pair_body(mm, carry):
        jA = wid + 32 * (2 * mm)
        jB = wid + 32 * (2 * mm + 1)

        @pl.when(jA < NCOL)
        def _():
            fire_reads(jA, win_a, sem_a)

        @pl.when(jB < NCOL)
        def _():
            fire_reads(jB, win_b, sem_b)

        @pl.when(jA < NCOL)
        def _():
            transpose(jA, win_a, stage_a, sem_a)

        @pl.when(jB < NCOL)
        def _():
            transpose(jB, win_b, stage_b, sem_b)

        @pl.when(jA < NCOL)
        def _():
            drain_write(stage_a)

        @pl.when(jB < NCOL)
        def _():
            drain_write(stage_b)
        return carry

    lax.fori_loop(0, (MPT + 1) // 2, pair_body, 0)

    @pl.when(wid == 0)
    def _():
        pltpu.sync_copy(tail_hbm, stage_a.at[pl.ds(0, TAILV // 2)])
        pltpu.sync_copy(stage_a.at[pl.ds(0, TAILV // 2)],
                        out_hbm.at[pl.ds((VOCAB - TAILV) // 2, TAILV // 2)])


def _gather_body(cs_hbm, os_hbm, word_hbm, bkp_hbm, out_hbm,
                 idx_os, idx_cs, idx_csh, idx_steps, bufs, cbuf, prow, ysig,
                 sem):
    wid = lax.axis_index("s") * NC + lax.axis_index("c")
    base = wid * BPW

    pltpu.sync_copy(cs_hbm.at[pl.ds(base, BPW)], idx_cs.at[pl.ds(0, BPW)])
    for c in range(CTX):
        pltpu.sync_copy(os_hbm.at[c, pl.ds(base, BPW)],
                        idx_os.at[c, pl.ds(0, BPW)])

    for q in range(BPW // 16):
        idx_csh[pl.ds(q * 16, 16)] = idx_cs[pl.ds(q * 16, 16)] >> 1
    for s in range(NSTEP):
        for c in range(CTX):
            for h in range(STEP // 16):
                v = idx_os[c, pl.ds(s * STEP + h * 16, 16)]
                idx_steps[s, pl.ds(c * STEP + h * 16, 16)] = v >> 1

    lane = lax.iota(jnp.int32, 16)

    def step(si, carry):
        sbase = si * STEP
        copies = []
        for g in range(NG):
            cp = pltpu.make_async_copy(
                word_hbm.at[idx_steps.at[si, pl.ds(g * 128, 128)]],
                bufs.at[pl.ds(g * 128, 128)], sem)
            cp.start()
            copies.append(cp)
        cpc = pltpu.make_async_copy(
            bkp_hbm.at[idx_csh.at[pl.ds(sbase, STEP)]], cbuf, sem)
        cpc.start()
        for cp in copies:
            cp.wait()
        cpc.wait()

        def row(r, rcarry):
            pr = jnp.zeros((16,), jnp.float32)
            cpar = (idx_cs[pl.ds(sbase + r, 16)][0] & 1) * 64
            pars = [(idx_os[c, pl.ds(sbase + r, 16)][0] & 1) * 64
                    for c in range(CTX)]
            for k in range(NK):
                a = bufs[r, pl.ds(pars[0] + k * 16, 16)]
                for c in range(1, CTX):
                    a = a + bufs[c * STEP + r, pl.ds(pars[c] + k * 16, 16)]
                pr = pr + a * cbuf[r, pl.ds(cpar + k * 16, 16)]
            prow[r] = pr * (1.0 / CTX)
            return rcarry

        lax.fori_loop(0, STEP, row, 0, unroll=2)

        for g in range(STEP // 16):
            y = jnp.zeros((16,), jnp.float32)
            for l in range(16):
                s = jnp.sum(prow[g * 16 + l])
                y = jnp.where(lane == l, s, y)
            ysig[pl.ds(sbase + g * 16, 16)] = 1.0 / (1.0 + jnp.exp(-y))
        return carry

    lax.fori_loop(0, NSTEP, step, 0)

    pltpu.sync_copy(ysig, out_hbm.at[pl.ds(base, BPW)])


@jax.jit
def _cbow(cs, os, word_embs, bkp_word_embs):
    mesh = plsc.VectorSubcoreMesh(core_axis_name="c", subcore_axis_name="s")
    relayout = pl.kernel(
        _relayout_body,
        out_type=jax.ShapeDtypeStruct((VOCAB // 2, 2 * DIM), jnp.float32),
        mesh=mesh,
        compiler_params=pltpu.CompilerParams(
            needs_layout_passes=False, use_tc_tiling_on_sc=True),
        scratch_types=[
            pltpu.VMEM((64, WP), jnp.float32),
            pltpu.VMEM((64, WP), jnp.float32),
            pltpu.VMEM((W // 2, 2 * DIM), jnp.float32),
            pltpu.VMEM((W // 2, 2 * DIM), jnp.float32),
            pltpu.SemaphoreType.DMA,
            pltpu.SemaphoreType.DMA,
            pltpu.SemaphoreType.DMA,
        ],
    )
    gather = pl.kernel(
        _gather_body,
        out_type=jax.ShapeDtypeStruct((BATCH,), jnp.float32),
        mesh=mesh,
        compiler_params=pltpu.CompilerParams(needs_layout_passes=False),
        scratch_types=[
            pltpu.VMEM((CTX, BPW + 16), jnp.int32),
            pltpu.VMEM((BPW + 16,), jnp.int32),
            pltpu.VMEM((BPW,), jnp.int32),
            pltpu.VMEM((NSTEP, CTX * STEP), jnp.int32),
            pltpu.VMEM((CTX * STEP, 2 * DIM), jnp.float32),
            pltpu.VMEM((STEP, 2 * DIM), jnp.float32),
            pltpu.VMEM((STEP, 16), jnp.float32),
            pltpu.VMEM((BPW,), jnp.float32),
            pltpu.SemaphoreType.DMA,
        ],
    )
    tail = word_embs[VOCAB - TAILV:].reshape(TAILV // 2, 2 * DIM)
    w2 = relayout(word_embs.T, tail)
    b2 = bkp_word_embs.reshape(VOCAB // 2, 2 * DIM)
    return gather(cs, os, w2, b2)


def kernel(cs, os, word_embs, bkp_word_embs):
    return _cbow(cs, os, word_embs, bkp_word_embs)


# speedup vs baseline: 1.7080x; 1.7080x over previous
import functools

import jax
import jax.numpy as jnp
from jax import lax
from jax.experimental import pallas as pl
from jax.experimental.pallas import tpu as pltpu
from jax.experimental.pallas import tpu_sc as plsc

VOCAB = 1000000
DIM = 64
BATCH = 16384
CTX = 20

NC = 2
NS = 16
NW = NC * NS
BPW = BATCH // NW
STEP = 64
NSTEP = BPW // STEP
NK = DIM // 16


def _body(cs_hbm, os_hbm, word_hbm, bkp_hbm, out_hbm,
          idx_os, idx_cs, bufs, cbuf, prow, ysig, sem):
    wid = lax.axis_index("s") * NC + lax.axis_index("c")
    base = wid * BPW

    pltpu.sync_copy(cs_hbm.at[pl.ds(base, BPW)], idx_cs)
    for c in range(CTX):
        pltpu.sync_copy(os_hbm.at[c, pl.ds(base, BPW)], idx_os.at[c])

    lane = lax.iota(jnp.int32, 16)

    def step(si, carry):
        sbase = si * STEP
        copies = []
        for c in range(CTX):
            cp = pltpu.make_async_copy(
                word_hbm.at[idx_os.at[c, pl.ds(sbase, STEP)]], bufs.at[c], sem)
            cp.start()
            copies.append(cp)
        cpc = pltpu.make_async_copy(
            bkp_hbm.at[idx_cs.at[pl.ds(sbase, STEP)]], cbuf, sem)
        cpc.start()
        for cp in copies:
            cp.wait()
        cpc.wait()

        def row(r, rcarry):
            pr = jnp.zeros((16,), jnp.float32)
            for k in range(NK):
                a = bufs[0, r, pl.ds(k * 16, 16)]
                for c in range(1, CTX):
                    a = a + bufs[c, r, pl.ds(k * 16, 16)]
                pr = pr + a * cbuf[r, pl.ds(k * 16, 16)]
            prow[r] = pr * (1.0 / CTX)
            return rcarry

        lax.fori_loop(0, STEP, row, 0, unroll=2)

        for g in range(STEP // 16):
            y = jnp.zeros((16,), jnp.float32)
            for l in range(16):
                s = jnp.sum(prow[g * 16 + l])
                y = jnp.where(lane == l, s, y)
            ysig[pl.ds(sbase + g * 16, 16)] = 1.0 / (1.0 + jnp.exp(-y))
        return carry

    lax.fori_loop(0, NSTEP, step, 0)

    pltpu.sync_copy(ysig, out_hbm.at[pl.ds(base, BPW)])


@jax.jit
def _cbow(cs, os, word_embs, bkp_word_embs):
    mesh = plsc.VectorSubcoreMesh(core_axis_name="c", subcore_axis_name="s")
    f = pl.kernel(
        _body,
        out_type=jax.ShapeDtypeStruct((BATCH,), jnp.float32),
        mesh=mesh,
        compiler_params=pltpu.CompilerParams(
            needs_layout_passes=False, use_tc_tiling_on_sc=False),
        scratch_types=[
            pltpu.VMEM((CTX, BPW), jnp.int32),
            pltpu.VMEM((BPW,), jnp.int32),
            pltpu.VMEM((CTX, STEP, DIM), jnp.float32),
            pltpu.VMEM((STEP, DIM), jnp.float32),
            pltpu.VMEM((STEP, 16), jnp.float32),
            pltpu.VMEM((BPW,), jnp.float32),
            pltpu.SemaphoreType.DMA,
        ],
    )
    return f(cs, os, word_embs, bkp_word_embs)


def kernel(cs, os, word_embs, bkp_word_embs):
    return _cbow(cs, os, word_embs, bkp_word_embs)
